# baseline (speedup 1.0000x reference)
import jax
import jax.numpy as jnp
from jax.experimental import pallas as pl


def _copy_body(x_ref, o_ref):
    o_ref[...] = x_ref[...]


def copy10(x):
    B, A, N, T = x.shape
    return pl.pallas_call(
        _copy_body,
        grid=(B,),
        in_specs=[pl.BlockSpec((1, A, N, T), lambda b: (b, 0, 0, 0))],
        out_specs=pl.BlockSpec((1, A, N, T), lambda b: (b, 0, 0, 0)),
        out_shape=jax.ShapeDtypeStruct(x.shape, x.dtype),
    )(x)


def kernel(p3, p4, p5, W1, b1, W2, b2, W3, b3):
    z = jnp.zeros((16, 3, 6400, 10), jnp.float32) + p3[0, 0, 0, 0]
    return (copy10(z), p4, p5)


# speedup vs baseline: 1.1047x; 1.1047x over previous
import functools

import jax
import jax.numpy as jnp
from jax.experimental import pallas as pl

_NA = 3
_NC = 10
_NO = _NA * _NC


def _mm_body(x_ref, w_ref, b_ref, o_ref):
    y = jax.lax.dot_general(
        w_ref[...], x_ref[0],
        dimension_numbers=(((1,), (0,)), ((), ())),
        preferred_element_type=jnp.float32,
    )
    o_ref[0] = y + b_ref[...]


def _head(x, W, b, n_tiles):
    B, C, H, Wd = x.shape
    hw = H * Wd
    t = hw // n_tiles
    xp = x.reshape(B, C, hw)
    z = pl.pallas_call(
        _mm_body,
        grid=(B, n_tiles),
        in_specs=[
            pl.BlockSpec((1, C, t), lambda bi, ti: (bi, 0, ti)),
            pl.BlockSpec((_NO, C), lambda bi, ti: (0, 0)),
            pl.BlockSpec((_NO, 1), lambda bi, ti: (0, 0)),
        ],
        out_specs=pl.BlockSpec((1, _NO, t), lambda bi, ti: (bi, 0, ti)),
        out_shape=jax.ShapeDtypeStruct((B, _NO, hw), jnp.float32),
    )(xp, W, b.reshape(_NO, 1))
    z = z.reshape(B, _NA, _NC, H, Wd)
    return jnp.transpose(z, (0, 1, 3, 4, 2))


def kernel(p3, p4, p5, W1, b1, W2, b2, W3, b3):
    o3 = _head(p3, W1, b1, 2)
    o4 = _head(p4, W2, b2, 1)
    o5 = _head(p5, W3, b3, 1)
    return (o3, o4, o5)
